# baseline (speedup 1.0000x reference)
import functools

import jax
import jax.numpy as jnp
from jax import lax
from jax.experimental import pallas as pl
from jax.experimental.pallas import tpu as pltpu
from jax.experimental.pallas import tpu_sc as plsc

NUM_EMB = 1_000_000
DIM = 32
NUM_HASH = 2
BATCH = 16384
NUM_CORES = 2
NUM_SUBCORES = 16
NUM_TILES = NUM_CORES * NUM_SUBCORES
B_PER_TILE = BATCH // NUM_TILES
CHUNK = 128
NCHUNK = B_PER_TILE // CHUNK
LANES = 16
NSTREAM = NUM_HASH * DIM * NCHUNK


def _body(tf_hbm, idx_hbm, hp_hbm, wb_hbm, outT_hbm,
          idx_v, h_v, gidx_v, g_v, outb_v, hp_v, wb_v, sem):
    wid = lax.axis_index("s") * NUM_CORES + lax.axis_index("c")
    base = wid * B_PER_TILE

    with jax.named_scope("copy_in"):
        pltpu.sync_copy(idx_hbm.at[pl.ds(base, B_PER_TILE)], idx_v)
        pltpu.sync_copy(hp_hbm, hp_v)
        pltpu.sync_copy(wb_hbm, wb_v)

    a0 = hp_v[pl.ds(0, LANES)]
    c0 = hp_v[pl.ds(16, LANES)]
    b0 = hp_v[pl.ds(32, LANES)]
    a1 = hp_v[pl.ds(48, LANES)]
    c1 = hp_v[pl.ds(64, LANES)]
    b1 = hp_v[pl.ds(80, LANES)]

    inv_m = jnp.float32(1.0 / NUM_EMB)
    m = jnp.int32(NUM_EMB)

    def _mod_m(s):
        q = (s.astype(jnp.float32) * inv_m).astype(jnp.int32)
        r = s - q * m
        r = jnp.where(r < 0, r + m, r)
        r = jnp.where(r >= m, r - m, r)
        return r

    def hash_iter(j, carry):
        off = pl.multiple_of(j * LANES, LANES)
        x = idx_v[pl.ds(off, LANES)]
        r = _mod_m(x)
        x1 = r >> 10
        x0 = r & 1023
        sl = pl.ds(off, LANES)
        h_v[0, sl] = _mod_m(x0 * a0 + x1 * c0 + b0)
        h_v[1, sl] = _mod_m(x0 * a1 + x1 * c1 + b1)
        return carry

    with jax.named_scope("hash"):
        lax.fori_loop(jnp.int32(0), jnp.int32(B_PER_TILE // LANES),
                      hash_iter, 0)

    with jax.named_scope("build_idx"):
        def idx_iter(r, carry):
            i = r >> 7
            col = (r >> 2) & 31
            k = r & 3
            off = (i * DIM + col) * m
            koff = k * CHUNK

            def grp(g, carry2):
                g16 = pl.multiple_of(g * LANES, LANES)
                gidx_v[r, pl.ds(g16, LANES)] = (
                    h_v[i, pl.ds(koff + g16, LANES)] + off)
                return carry2

            lax.fori_loop(jnp.int32(0), jnp.int32(CHUNK // LANES), grp, 0)
            return carry

        lax.fori_loop(jnp.int32(0), jnp.int32(NSTREAM), idx_iter, 0)

    with jax.named_scope("gather"):
        def fire(r, carry):
            pltpu.async_copy(tf_hbm.at[gidx_v.at[r]], g_v.at[r], sem)
            return carry

        lax.fori_loop(jnp.int32(0), jnp.int32(NSTREAM), fire, 0)

        def drain(r, carry):
            pltpu.make_async_copy(
                tf_hbm.at[gidx_v.at[r]], g_v.at[r], sem).wait()
            return carry

        lax.fori_loop(jnp.int32(0), jnp.int32(NSTREAM), drain, 0)

    w0 = wb_v[pl.ds(0, LANES)]
    w1 = wb_v[pl.ds(16, LANES)]

    with jax.named_scope("combine"):
        def comb_iter(r, carry):
            col = r >> 2
            k = r & 3
            koff = k * CHUNK

            def grp(g, carry2):
                g16 = pl.multiple_of(g * LANES, LANES)
                sl = pl.ds(g16, LANES)
                outb_v[col, pl.ds(koff + g16, LANES)] = (
                    g_v[r, sl] * w0 + g_v[r + NSTREAM // 2, sl] * w1)
                return carry2

            lax.fori_loop(jnp.int32(0), jnp.int32(CHUNK // LANES), grp, 0)
            return carry

        lax.fori_loop(jnp.int32(0), jnp.int32(NSTREAM // 2), comb_iter, 0)

    with jax.named_scope("copy_out"):
        for c in range(DIM):
            pltpu.sync_copy(outb_v.at[jnp.int32(c)],
                            outT_hbm.at[jnp.int32(c), pl.ds(base, B_PER_TILE)])


def kernel(indices, tables, weights, hash_a, hash_b):
    idx32 = indices.astype(jnp.int32)
    tflat = jnp.transpose(tables, (0, 2, 1)).reshape(
        NUM_HASH * DIM * NUM_EMB)
    a_mod = jnp.mod(hash_a, NUM_EMB).astype(jnp.int32)
    b_mod = jnp.mod(hash_b, NUM_EMB).astype(jnp.int32)
    c_mod = jnp.mod(a_mod * 1024, NUM_EMB).astype(jnp.int32)
    hp = jnp.stack([a_mod[0], c_mod[0], b_mod[0],
                    a_mod[1], c_mod[1], b_mod[1]]).astype(jnp.int32)
    hp = jnp.broadcast_to(hp[:, None], (6, LANES)).reshape(6 * LANES)
    wb = jnp.broadcast_to(
        weights.astype(jnp.float32)[:, None], (2, LANES)).reshape(2 * LANES)

    mesh = plsc.VectorSubcoreMesh(
        core_axis_name="c", subcore_axis_name="s")
    run = pl.kernel(
        _body,
        out_type=jax.ShapeDtypeStruct((DIM, BATCH), jnp.float32),
        mesh=mesh,
        scratch_types=[
            pltpu.VMEM((B_PER_TILE,), jnp.int32),
            pltpu.VMEM((NUM_HASH, B_PER_TILE), jnp.int32),
            pltpu.VMEM((NSTREAM, CHUNK), jnp.int32),
            pltpu.VMEM((NSTREAM, CHUNK), jnp.float32),
            pltpu.VMEM((DIM, B_PER_TILE), jnp.float32),
            pltpu.VMEM((6 * LANES,), jnp.int32),
            pltpu.VMEM((2 * LANES,), jnp.float32),
            pltpu.SemaphoreType.DMA,
        ],
        compiler_params=pltpu.CompilerParams(
            needs_layout_passes=False),
    )
    return run(tflat, idx32, hp, wb).T


# speedup vs baseline: 5.3809x; 5.3809x over previous
import functools

import jax
import jax.numpy as jnp
from jax import lax
from jax.experimental import pallas as pl
from jax.experimental.pallas import tpu as pltpu
from jax.experimental.pallas import tpu_sc as plsc

NUM_EMB = 1_000_000
DIM = 32
BATCH = 16384
NUM_CORES = 2
NUM_SUBCORES = 16
NUM_TILES = NUM_CORES * NUM_SUBCORES
B_PER_TILE = BATCH // NUM_TILES
CHUNK = 128
NCHUNK = B_PER_TILE // CHUNK
LANES = 16
WIDE = 128
EMB_PER_WIDE = WIDE // DIM
V_WIDE = NUM_EMB // EMB_PER_WIDE
OUT_ROWS_PER_TILE = B_PER_TILE * DIM // WIDE


def _body(tw_hbm, idx_hbm, hp_hbm, wb_hbm, out_hbm,
          idx_v, h_v, off_v, r0_v, r1_v, outb_v, hp_v, wb_v, sem0, sem1):
    t0_hbm = tw_hbm.at[jnp.int32(0)]
    t1_hbm = tw_hbm.at[jnp.int32(1)]
    wid = lax.axis_index("s") * NUM_CORES + lax.axis_index("c")
    base = wid * B_PER_TILE

    with jax.named_scope("copy_in"):
        pltpu.sync_copy(idx_hbm.at[pl.ds(base, B_PER_TILE)], idx_v)
        pltpu.sync_copy(hp_hbm, hp_v)
        pltpu.sync_copy(wb_hbm, wb_v)

    a0 = hp_v[pl.ds(0, LANES)]
    c0 = hp_v[pl.ds(16, LANES)]
    b0 = hp_v[pl.ds(32, LANES)]
    a1 = hp_v[pl.ds(48, LANES)]
    c1 = hp_v[pl.ds(64, LANES)]
    b1 = hp_v[pl.ds(80, LANES)]

    inv_m = jnp.float32(1.0 / NUM_EMB)
    m = jnp.int32(NUM_EMB)

    def _mod_m(s):
        q = (s.astype(jnp.float32) * inv_m).astype(jnp.int32)
        r = s - q * m
        r = jnp.where(r < 0, r + m, r)
        r = jnp.where(r >= m, r - m, r)
        return r

    def hash_iter(j, carry):
        off = pl.multiple_of(j * LANES, LANES)
        x = idx_v[pl.ds(off, LANES)]
        r = _mod_m(x)
        x1 = r >> 10
        x0 = r & 1023
        chunk = j >> 3
        lane_off = pl.multiple_of((j & 7) * LANES, LANES)
        h0 = _mod_m(x0 * a0 + x1 * c0 + b0)
        h1 = _mod_m(x0 * a1 + x1 * c1 + b1)
        sl = pl.ds(lane_off, LANES)
        h_v[0, chunk, sl] = h0 >> 2
        h_v[1, chunk, sl] = h1 >> 2
        off_v[0, chunk, sl] = (h0 & 3) << 5
        off_v[1, chunk, sl] = (h1 & 3) << 5
        return carry

    with jax.named_scope("hash"):
        lax.fori_loop(jnp.int32(0), jnp.int32(B_PER_TILE // LANES),
                      hash_iter, 0)

    w0 = wb_v[pl.ds(0, LANES)]
    w1 = wb_v[pl.ds(16, LANES)]

    def fire(j, sem):
        jj = jnp.int32(j)
        b = jnp.int32(j & 1)
        return (
            pltpu.async_copy(
                t0_hbm.at[h_v.at[jnp.int32(0), jj]], r0_v.at[b], sem),
            pltpu.async_copy(
                t1_hbm.at[h_v.at[jnp.int32(1), jj]], r1_v.at[b], sem),
        )

    def combine(j):
        b = jnp.int32(j & 1)
        r0b = r0_v.at[b]
        r1b = r1_v.at[b]
        jj = jnp.int32(j)

        def grp_iter(g, carry):
            g16 = g * LANES
            rel = lax.iota(jnp.int32, 16) + g16
            o0 = off_v[0, jj, pl.ds(pl.multiple_of(g16, LANES), LANES)]
            o1 = off_v[1, jj, pl.ds(pl.multiple_of(g16, LANES), LANES)]
            k = rel + jnp.int32(j * CHUNK)
            orow = k >> 2
            ocol = (rel & 3) << 5

            def col_iter(c, carry2):
                cb = jnp.broadcast_to(c, (LANES,))
                v0 = plsc.load_gather(r0b, [rel, o0 + cb])
                v1 = plsc.load_gather(r1b, [rel, o1 + cb])
                plsc.store_scatter(outb_v, [orow, ocol + cb],
                                   v0 * w0 + v1 * w1)
                return carry2

            lax.fori_loop(jnp.int32(0), jnp.int32(DIM), col_iter, 0)
            return carry

        lax.fori_loop(jnp.int32(0), jnp.int32(CHUNK // LANES), grp_iter, 0)

    sems = (sem0, sem1)
    with jax.named_scope("gather_combine"):
        pend = fire(0, sems[0])
        for j in range(NCHUNK):
            nxt = fire(j + 1, sems[(j + 1) & 1]) if j + 1 < NCHUNK else None
            for h in pend:
                h.wait()
            combine(j)
            pend = nxt

    with jax.named_scope("copy_out"):
        pltpu.sync_copy(
            outb_v,
            out_hbm.at[pl.ds(wid * OUT_ROWS_PER_TILE, OUT_ROWS_PER_TILE)])


def kernel(indices, tables, weights, hash_a, hash_b):
    idx32 = indices.astype(jnp.int32)
    tw = (jnp.transpose(tables, (0, 2, 1))
          .reshape(2, DIM, V_WIDE, EMB_PER_WIDE)
          .transpose(0, 2, 3, 1)
          .reshape(2, V_WIDE, WIDE))
    a_mod = jnp.mod(hash_a, NUM_EMB).astype(jnp.int32)
    b_mod = jnp.mod(hash_b, NUM_EMB).astype(jnp.int32)
    c_mod = jnp.mod(a_mod * 1024, NUM_EMB).astype(jnp.int32)
    hp = jnp.stack([a_mod[0], c_mod[0], b_mod[0],
                    a_mod[1], c_mod[1], b_mod[1]]).astype(jnp.int32)
    hp = jnp.broadcast_to(hp[:, None], (6, LANES)).reshape(6 * LANES)
    wb = jnp.broadcast_to(
        weights.astype(jnp.float32)[:, None], (2, LANES)).reshape(2 * LANES)

    mesh = plsc.VectorSubcoreMesh(
        core_axis_name="c", subcore_axis_name="s")
    run = pl.kernel(
        _body,
        out_type=jax.ShapeDtypeStruct((BATCH * DIM // WIDE, WIDE),
                                      jnp.float32),
        mesh=mesh,
        scratch_types=[
            pltpu.VMEM((B_PER_TILE,), jnp.int32),
            pltpu.VMEM((2, NCHUNK, CHUNK), jnp.int32),
            pltpu.VMEM((2, NCHUNK, CHUNK), jnp.int32),
            pltpu.VMEM((2, CHUNK, WIDE), jnp.float32),
            pltpu.VMEM((2, CHUNK, WIDE), jnp.float32),
            pltpu.VMEM((OUT_ROWS_PER_TILE, WIDE), jnp.float32),
            pltpu.VMEM((6 * LANES,), jnp.int32),
            pltpu.VMEM((2 * LANES,), jnp.float32),
            pltpu.SemaphoreType.DMA,
            pltpu.SemaphoreType.DMA,
        ],
        compiler_params=pltpu.CompilerParams(
            use_tc_tiling_on_sc=True, needs_layout_passes=False),
    )
    return run(tw, idx32, hp, wb).reshape(BATCH, DIM)
